# baseline (speedup 1.0000x reference)
import math

import jax
import jax.numpy as jnp
from jax import lax
from jax.experimental import pallas as pl
from jax.experimental.pallas import tpu as pltpu
from jax.experimental.pallas import tpu_sc as plsc

DIM = 128
B = 1024
L = 200
N_TOK = B * L
NC, NS = 2, 16
NW = NC * NS
PER_W = N_TOK // NW
CHUNK = 40
N_CHUNKS = PER_W // CHUNK
SCALE = math.sqrt(DIM)


def _sc_embed(idx_flat, lut, pe2d):
    mesh = plsc.VectorSubcoreMesh(core_axis_name="c", subcore_axis_name="s")

    def body(idx_hbm, lut_hbm, pe_hbm, out_hbm, idx_v, pe_v,
             gb0, gb1, gsem0, gsem1, ssem0, ssem1):
        gb = (gb0, gb1)
        gsem = (gsem0, gsem1)
        ssem = (ssem0, ssem1)
        wid = lax.axis_index("s") * NC + lax.axis_index("c")
        base = wid * PER_W
        pltpu.sync_copy(idx_hbm.at[pl.ds(base, PER_W)], idx_v)
        pltpu.sync_copy(pe_hbm, pe_v)

        def start_gather(j, b):
            pltpu.async_copy(
                lut_hbm.at[idx_v.at[pl.ds(j * CHUNK, CHUNK)]], gb[b], gsem[b])

        def wait_gather(b):
            pltpu.make_async_copy(
                lut_hbm.at[idx_v.at[pl.ds(0, CHUNK)]], gb[b], gsem[b]).wait()

        def start_scatter(j, b):
            pltpu.async_copy(
                gb[b], out_hbm.at[pl.ds(base + j * CHUNK, CHUNK)], ssem[b])

        def wait_scatter(b):
            pltpu.make_async_copy(
                gb[b], out_hbm.at[pl.ds(base, CHUNK)], ssem[b]).wait()

        def compute(j, b):
            ph = lax.rem(j * CHUNK, L)
            buf = gb[b]

            @plsc.parallel_loop(0, CHUNK, unroll=4)
            def _(r):
                for v in range(DIM // 16):
                    sl = pl.ds(v * 16, 16)
                    buf[r, sl] = buf[r, sl] * SCALE + pe_v[ph + r, sl]

        start_gather(0, 0)

        def pair_body(j2, carry):
            for b in (0, 1):
                j = j2 * 2 + b
                nb = 1 - b
                @pl.when(j >= 1)
                def _():
                    wait_scatter(nb)

                @pl.when(j + 1 < N_CHUNKS)
                def _():
                    start_gather(j + 1, nb)

                wait_gather(b)
                compute(j, b)
                start_scatter(j, b)
            return carry

        lax.fori_loop(0, N_CHUNKS // 2, pair_body, 0)
        wait_scatter(1)

    run = pl.kernel(
        body,
        out_type=jax.ShapeDtypeStruct((N_TOK, DIM), jnp.float32),
        mesh=mesh,
        scratch_types=[
            pltpu.VMEM((PER_W,), jnp.int32),
            pltpu.VMEM((L, DIM), jnp.float32),
            pltpu.VMEM((CHUNK, DIM), jnp.float32),
            pltpu.VMEM((CHUNK, DIM), jnp.float32),
            pltpu.SemaphoreType.DMA,
            pltpu.SemaphoreType.DMA,
            pltpu.SemaphoreType.DMA,
            pltpu.SemaphoreType.DMA,
        ],
    )
    return run(idx_flat, lut, pe2d)


@jax.jit
def kernel(tensor, lut, pe):
    idx_flat = tensor.reshape(N_TOK)
    pe2d = pe[0, :L, :]
    out = _sc_embed(idx_flat, lut, pe2d)
    return out.reshape(B, L, DIM)


# speedup vs baseline: 1.2694x; 1.2694x over previous
import math

import jax
import jax.numpy as jnp
from jax import lax
from jax.experimental import pallas as pl
from jax.experimental.pallas import tpu as pltpu
from jax.experimental.pallas import tpu_sc as plsc

DIM = 128
B = 1024
L = 200
N_TOK = B * L
NC, NS = 2, 16
NW = NC * NS
PER_W = N_TOK // NW
CHUNK = 40
N_CHUNKS = PER_W // CHUNK
SCALE = math.sqrt(DIM)


def _sc_embed(idx_flat, lut, pe2d):
    mesh = plsc.VectorSubcoreMesh(core_axis_name="c", subcore_axis_name="s")

    def body(idx_hbm, lut_hbm, pe_hbm, out_hbm, idx_v, pe_v,
             gb0, gb1, gb2, gb3, gsem0, gsem1, gsem2, gsem3,
             ssem0, ssem1, ssem2, ssem3):
        gb = (gb0, gb1, gb2, gb3)
        gsem = (gsem0, gsem1, gsem2, gsem3)
        ssem = (ssem0, ssem1, ssem2, ssem3)
        wid = lax.axis_index("s") * NC + lax.axis_index("c")
        base = wid * PER_W
        pltpu.sync_copy(idx_hbm.at[pl.ds(base, PER_W)], idx_v)
        pltpu.sync_copy(pe_hbm, pe_v)

        def start_gather(j, b):
            pltpu.async_copy(
                lut_hbm.at[idx_v.at[pl.ds(j * CHUNK, CHUNK)]], gb[b], gsem[b])

        def wait_gather(b):
            pltpu.make_async_copy(
                lut_hbm.at[idx_v.at[pl.ds(0, CHUNK)]], gb[b], gsem[b]).wait()

        def start_scatter(j, b):
            pltpu.async_copy(
                gb[b], out_hbm.at[pl.ds(base + j * CHUNK, CHUNK)], ssem[b])

        def wait_scatter(b):
            pltpu.make_async_copy(
                gb[b], out_hbm.at[pl.ds(base, CHUNK)], ssem[b]).wait()

        def compute(j, b):
            ph = lax.rem(j * CHUNK, L)
            buf = gb[b]

            @plsc.parallel_loop(0, CHUNK, unroll=8)
            def _(r):
                for v in range(DIM // 16):
                    sl = pl.ds(v * 16, 16)
                    buf[r, sl] = buf[r, sl] * SCALE + pe_v[ph + r, sl]

        start_gather(0, 0)
        start_gather(1, 1)
        start_gather(2, 2)

        def quad_body(j4, carry):
            for b in range(4):
                j = j4 * 4 + b
                nb = (b + 3) % 4
                @pl.when(j >= 1)
                def _():
                    wait_scatter(nb)

                @pl.when(j + 3 < N_CHUNKS)
                def _():
                    start_gather(j + 3, nb)

                wait_gather(b)
                compute(j, b)
                start_scatter(j, b)
            return carry

        lax.fori_loop(0, N_CHUNKS // 4, quad_body, 0)
        wait_scatter((N_CHUNKS - 1) % 4)

    run = pl.kernel(
        body,
        out_type=jax.ShapeDtypeStruct((N_TOK, DIM), jnp.float32),
        mesh=mesh,
        scratch_types=[
            pltpu.VMEM((PER_W,), jnp.int32),
            pltpu.VMEM((L, DIM), jnp.float32),
            pltpu.VMEM((CHUNK, DIM), jnp.float32),
            pltpu.VMEM((CHUNK, DIM), jnp.float32),
            pltpu.VMEM((CHUNK, DIM), jnp.float32),
            pltpu.VMEM((CHUNK, DIM), jnp.float32),
            pltpu.SemaphoreType.DMA,
            pltpu.SemaphoreType.DMA,
            pltpu.SemaphoreType.DMA,
            pltpu.SemaphoreType.DMA,
            pltpu.SemaphoreType.DMA,
            pltpu.SemaphoreType.DMA,
            pltpu.SemaphoreType.DMA,
            pltpu.SemaphoreType.DMA,
        ],
    )
    return run(idx_flat, lut, pe2d)


@jax.jit
def kernel(tensor, lut, pe):
    idx_flat = tensor.reshape(N_TOK)
    pe2d = pe[0, :L, :]
    out = _sc_embed(idx_flat, lut, pe2d)
    return out.reshape(B, L, DIM)


# speedup vs baseline: 1.3063x; 1.0290x over previous
import math

import jax
import jax.numpy as jnp
from jax import lax
from jax.experimental import pallas as pl
from jax.experimental.pallas import tpu as pltpu
from jax.experimental.pallas import tpu_sc as plsc

DIM = 128
B = 1024
L = 200
N_TOK = B * L
NC, NS = 2, 16
NW = NC * NS
PER_W = N_TOK // NW
CHUNK = 64
N_CHUNKS = PER_W // CHUNK
SCALE = math.sqrt(DIM)


def _sc_embed(idx_flat, lut, pe2d):
    mesh = plsc.VectorSubcoreMesh(core_axis_name="c", subcore_axis_name="s")

    def body(idx_hbm, lut_hbm, pe_hbm, out_hbm, idx_v, pe_v,
             gb0, gb1, gb2, gb3, gsem0, gsem1, gsem2, gsem3,
             ssem0, ssem1, ssem2, ssem3):
        gb = (gb0, gb1, gb2, gb3)
        gsem = (gsem0, gsem1, gsem2, gsem3)
        ssem = (ssem0, ssem1, ssem2, ssem3)
        wid = lax.axis_index("s") * NC + lax.axis_index("c")
        base = wid * PER_W
        pltpu.sync_copy(idx_hbm.at[pl.ds(base, PER_W)], idx_v)
        pltpu.sync_copy(pe_hbm, pe_v)

        def start_gather(j, b):
            pltpu.async_copy(
                lut_hbm.at[idx_v.at[pl.ds(j * CHUNK, CHUNK)]], gb[b], gsem[b])

        def wait_gather(b):
            pltpu.make_async_copy(
                lut_hbm.at[idx_v.at[pl.ds(0, CHUNK)]], gb[b], gsem[b]).wait()

        def start_scatter(j, b):
            pltpu.async_copy(
                gb[b], out_hbm.at[pl.ds(base + j * CHUNK, CHUNK)], ssem[b])

        def wait_scatter(b):
            pltpu.make_async_copy(
                gb[b], out_hbm.at[pl.ds(base, CHUNK)], ssem[b]).wait()

        def compute(j, b):
            ph = lax.rem(j * CHUNK, L)
            buf = gb[b]

            @plsc.parallel_loop(0, CHUNK, unroll=8)
            def _(r):
                lrow = ph + r
                lrow = jnp.where(lrow >= L, lrow - L, lrow)
                for v in range(DIM // 16):
                    sl = pl.ds(v * 16, 16)
                    buf[r, sl] = buf[r, sl] * SCALE + pe_v[lrow, sl]

        start_gather(0, 0)
        start_gather(1, 1)
        start_gather(2, 2)

        def quad_body(j4, carry):
            for b in range(4):
                j = j4 * 4 + b
                nb = (b + 3) % 4
                @pl.when(j >= 1)
                def _():
                    wait_scatter(nb)

                @pl.when(j + 3 < N_CHUNKS)
                def _():
                    start_gather(j + 3, nb)

                wait_gather(b)
                compute(j, b)
                start_scatter(j, b)
            return carry

        lax.fori_loop(0, N_CHUNKS // 4, quad_body, 0)
        wait_scatter((N_CHUNKS - 1) % 4)

    run = pl.kernel(
        body,
        out_type=jax.ShapeDtypeStruct((N_TOK, DIM), jnp.float32),
        mesh=mesh,
        scratch_types=[
            pltpu.VMEM((PER_W,), jnp.int32),
            pltpu.VMEM((L, DIM), jnp.float32),
            pltpu.VMEM((CHUNK, DIM), jnp.float32),
            pltpu.VMEM((CHUNK, DIM), jnp.float32),
            pltpu.VMEM((CHUNK, DIM), jnp.float32),
            pltpu.VMEM((CHUNK, DIM), jnp.float32),
            pltpu.SemaphoreType.DMA,
            pltpu.SemaphoreType.DMA,
            pltpu.SemaphoreType.DMA,
            pltpu.SemaphoreType.DMA,
            pltpu.SemaphoreType.DMA,
            pltpu.SemaphoreType.DMA,
            pltpu.SemaphoreType.DMA,
            pltpu.SemaphoreType.DMA,
        ],
    )
    return run(idx_flat, lut, pe2d)


@jax.jit
def kernel(tensor, lut, pe):
    idx_flat = tensor.reshape(N_TOK)
    pe2d = pe[0, :L, :]
    out = _sc_embed(idx_flat, lut, pe2d)
    return out.reshape(B, L, DIM)


# speedup vs baseline: 1.3093x; 1.0023x over previous
import math

import jax
import jax.numpy as jnp
from jax import lax
from jax.experimental import pallas as pl
from jax.experimental.pallas import tpu as pltpu
from jax.experimental.pallas import tpu_sc as plsc

DIM = 128
B = 1024
L = 200
N_TOK = B * L
NC, NS = 2, 16
NW = NC * NS
PER_W = N_TOK // NW
CHUNK = 128
N_CHUNKS = PER_W // CHUNK
NBUF = 5
UNROLL = 8
SCALE = math.sqrt(DIM)
assert N_CHUNKS % NBUF == 0


def _sc_embed(idx_flat, lut, pe2d):
    mesh = plsc.VectorSubcoreMesh(core_axis_name="c", subcore_axis_name="s")

    def body(idx_hbm, lut_hbm, pe_hbm, out_hbm, *scratch):
        idx_v, pe_v = scratch[0], scratch[1]
        gb = scratch[2:2 + NBUF]
        gsem = scratch[2 + NBUF:2 + 2 * NBUF]
        ssem = scratch[2 + 2 * NBUF:2 + 3 * NBUF]
        wid = lax.axis_index("s") * NC + lax.axis_index("c")
        base = wid * PER_W
        pltpu.sync_copy(idx_hbm.at[pl.ds(base, PER_W)], idx_v)
        pltpu.sync_copy(pe_hbm, pe_v)

        def start_gather(j, b):
            pltpu.async_copy(
                lut_hbm.at[idx_v.at[pl.ds(j * CHUNK, CHUNK)]], gb[b], gsem[b])

        def wait_gather(b):
            pltpu.make_async_copy(
                lut_hbm.at[idx_v.at[pl.ds(0, CHUNK)]], gb[b], gsem[b]).wait()

        def start_scatter(j, b):
            pltpu.async_copy(
                gb[b], out_hbm.at[pl.ds(base + j * CHUNK, CHUNK)], ssem[b])

        def wait_scatter(b):
            pltpu.make_async_copy(
                gb[b], out_hbm.at[pl.ds(base, CHUNK)], ssem[b]).wait()

        def compute(j, b):
            ph = lax.rem(j * CHUNK, L)
            buf = gb[b]

            @plsc.parallel_loop(0, CHUNK, unroll=UNROLL)
            def _(r):
                lrow = ph + r
                lrow = jnp.where(lrow >= L, lrow - L, lrow)
                for v in range(DIM // 16):
                    sl = pl.ds(v * 16, 16)
                    buf[r, sl] = buf[r, sl] * SCALE + pe_v[lrow, sl]

        for j in range(NBUF - 1):
            start_gather(j, j)

        def ring_body(jr, carry):
            for b in range(NBUF):
                j = jr * NBUF + b
                nb = (b + NBUF - 1) % NBUF
                @pl.when(j >= 1)
                def _():
                    wait_scatter(nb)

                @pl.when(j + NBUF - 1 < N_CHUNKS)
                def _():
                    start_gather(j + NBUF - 1, nb)

                wait_gather(b)
                compute(j, b)
                start_scatter(j, b)
            return carry

        lax.fori_loop(0, N_CHUNKS // NBUF, ring_body, 0)
        wait_scatter((N_CHUNKS - 1) % NBUF)

    run = pl.kernel(
        body,
        out_type=jax.ShapeDtypeStruct((N_TOK, DIM), jnp.float32),
        mesh=mesh,
        scratch_types=(
            [pltpu.VMEM((PER_W,), jnp.int32),
             pltpu.VMEM((L, DIM), jnp.float32)]
            + [pltpu.VMEM((CHUNK, DIM), jnp.float32)] * NBUF
            + [pltpu.SemaphoreType.DMA] * (2 * NBUF)
        ),
    )
    return run(idx_flat, lut, pe2d)


@jax.jit
def kernel(tensor, lut, pe):
    idx_flat = tensor.reshape(N_TOK)
    pe2d = pe[0, :L, :]
    out = _sc_embed(idx_flat, lut, pe2d)
    return out.reshape(B, L, DIM)
